# baseline (speedup 1.0000x reference)
import functools

import jax
import jax.numpy as jnp
from jax import lax
from jax.experimental import pallas as pl
from jax.experimental.pallas import tpu as pltpu
from jax.experimental.pallas import tpu_sc as plsc

N = 10000
G = 128
E = 160000
NACC = 10240
NTILES = 32
BB = 56
NB = 96
NBUF = 2
T_EDGE = NB * BB
EPAD = NTILES * T_EDGE
DUMMY = N
RB = 1000
NRB = N // RB
RPT = NACC // 16
ACCN = 10112
RPTA = ACCN // 16


def _p1(x, W, a_s, a_d, H, O):
    K = x.shape[1]
    HO = H * O

    def body(x_ref, w_ref, as_ref, ad_ref, h_ref, es_ref):
        hb = jnp.dot(x_ref[...], w_ref[...], preferred_element_type=jnp.float32)
        h_ref[...] = hb
        cols = []
        for hh in range(H):
            hs = hb[:, hh * O:(hh + 1) * O]
            cols.append(jnp.sum(hs * as_ref[hh, :][None, :], axis=1))
        for hh in range(H):
            hs = hb[:, hh * O:(hh + 1) * O]
            cols.append(jnp.sum(hs * ad_ref[hh, :][None, :], axis=1))
        es_ref[...] = jnp.stack(cols, axis=1)

    return pl.pallas_call(
        body,
        grid=(NRB,),
        in_specs=[
            pl.BlockSpec((RB, K), lambda r: (r, 0)),
            pl.BlockSpec((K, HO), lambda r: (0, 0)),
            pl.BlockSpec((H, O), lambda r: (0, 0)),
            pl.BlockSpec((H, O), lambda r: (0, 0)),
        ],
        out_specs=[
            pl.BlockSpec((RB, HO), lambda r: (r, 0)),
            pl.BlockSpec((RB, 2 * H), lambda r: (r, 0)),
        ],
        out_shape=[
            jax.ShapeDtypeStruct((N, HO), jnp.float32),
            jax.ShapeDtypeStruct((N, 2 * H), jnp.float32),
        ],
    )(x, W, a_s, a_d)


def _p1l0(x, W, a_s, a_d, H, O):
    def body(x_ref, w_ref, as_ref, ad_ref, es_ref):
        xb = x_ref[...]
        cols = []
        for hh in range(H):
            vs = jnp.dot(w_ref[:, hh * O:(hh + 1) * O], as_ref[hh, :],
                         preferred_element_type=jnp.float32)
            cols.append(jnp.dot(xb, vs, preferred_element_type=jnp.float32))
        for hh in range(H):
            vd = jnp.dot(w_ref[:, hh * O:(hh + 1) * O], ad_ref[hh, :],
                         preferred_element_type=jnp.float32)
            cols.append(jnp.dot(xb, vd, preferred_element_type=jnp.float32))
        es_ref[...] = jnp.stack(cols, axis=1)

    return pl.pallas_call(
        body,
        grid=(NRB,),
        in_specs=[
            pl.BlockSpec((RB, G), lambda r: (r, 0)),
            pl.BlockSpec((G, H * O), lambda r: (0, 0)),
            pl.BlockSpec((H, O), lambda r: (0, 0)),
            pl.BlockSpec((H, O), lambda r: (0, 0)),
        ],
        out_specs=pl.BlockSpec((RB, 2 * H), lambda r: (r, 0)),
        out_shape=jax.ShapeDtypeStruct((N, 2 * H), jnp.float32),
    )(x, W, a_s, a_d)


def _p4l0(op_stack, rden4, W, b_row, H, O):
    def body(op_ref, rd_ref, w_ref, b_ref, y_ref, s_ref):
        r = pl.program_id(1)
        agg = op_ref[0, 0] + op_ref[0, 1]
        z = agg * rd_ref[0, 0, 0, :][:, None]
        y = jnp.dot(z, w_ref[0], preferred_element_type=jnp.float32) + b_ref[0, :][None, :]
        y_ref[...] = y
        su = jnp.sum(y, axis=0)
        sq = jnp.sum(y * y, axis=0)
        stat = jnp.concatenate(
            [su[None, :], sq[None, :], jnp.zeros((6, O), jnp.float32)], axis=0)

        @pl.when(r == 0)
        def _():
            s_ref[...] = stat

        @pl.when(r > 0)
        def _():
            s_ref[...] = s_ref[...] + stat

    return pl.pallas_call(
        body,
        grid=(H, NRB),
        in_specs=[
            pl.BlockSpec((1, 2, RB, G), lambda h, r: (h, 0, r, 0)),
            pl.BlockSpec((1, 1, 1, RB), lambda h, r: (h, r, 0, 0)),
            pl.BlockSpec((1, G, O), lambda h, r: (h, 0, 0)),
            pl.BlockSpec((1, O), lambda h, r: (0, h)),
        ],
        out_specs=[
            pl.BlockSpec((RB, O), lambda h, r: (r, h)),
            pl.BlockSpec((8, O), lambda h, r: (0, h)),
        ],
        out_shape=[
            jax.ShapeDtypeStruct((N, H * O), jnp.float32),
            jax.ShapeDtypeStruct((8, H * O), jnp.float32),
        ],
    )(op_stack, rden4, W.reshape(G, H, O).transpose(1, 0, 2), b_row)


@functools.lru_cache(maxsize=None)
def _make_p2(H):
    mesh = plsc.VectorSubcoreMesh(core_axis_name="c", subcore_axis_name="s")

    @functools.partial(
        pl.kernel, mesh=mesh,
        compiler_params=pltpu.CompilerParams(needs_layout_passes=False),
        out_type=(jax.ShapeDtypeStruct((H * EPAD,), jnp.float32),
                  jax.ShapeDtypeStruct((NTILES * H * NACC,), jnp.float32)),
        scratch_types=[pltpu.VMEM((T_EDGE,), jnp.int32),
                       pltpu.VMEM((T_EDGE,), jnp.int32),
                       pltpu.VMEM((NACC,), jnp.float32),
                       pltpu.VMEM((NACC,), jnp.float32),
                       pltpu.VMEM((T_EDGE,), jnp.float32),
                       pltpu.VMEM((NACC,), jnp.float32)],
    )
    def p2(est, edt, src_all, dst_all, w_out, den_out,
           src_v, dst_v, es_v, ed_v, w_v, den_v):
        wid = lax.axis_index("c") * 16 + lax.axis_index("s")
        pltpu.sync_copy(src_all.at[pl.ds(wid * T_EDGE, T_EDGE)], src_v)
        pltpu.sync_copy(dst_all.at[pl.ds(wid * T_EDGE, T_EDGE)], dst_v)
        for hh in range(H):
            pltpu.sync_copy(est.at[pl.ds(hh * NACC, NACC)], es_v)
            pltpu.sync_copy(edt.at[pl.ds(hh * NACC, NACC)], ed_v)

            def zbody(i, _):
                den_v[pl.ds(i * 16, 16)] = jnp.zeros((16,), jnp.float32)
                return 0
            lax.fori_loop(0, NACC // 16, zbody, 0)

            def ebody(g, _):
                sl = pl.ds(g * 16, 16)
                s16 = src_v[sl]
                d16 = dst_v[sl]
                esg = plsc.load_gather(es_v, [s16])
                edg = plsc.load_gather(ed_v, [d16])
                e = esg + edg
                e = jnp.where(e >= 0.0, e, 0.2 * e)
                wv = jnp.exp(e)
                w_v[sl] = wv
                plsc.addupdate_scatter(den_v, [d16], wv)
                return 0
            lax.fori_loop(0, T_EDGE // 16, ebody, 0)

            pltpu.sync_copy(w_v, w_out.at[pl.ds(hh * EPAD + wid * T_EDGE, T_EDGE)])
            pltpu.sync_copy(den_v, den_out.at[pl.ds((wid * H + hh) * NACC, NACC)])

    return p2


@functools.lru_cache(maxsize=None)
def _make_p3():
    mesh = plsc.VectorSubcoreMesh(core_axis_name="c", subcore_axis_name="s")

    @functools.partial(
        pl.kernel, mesh=mesh,
        compiler_params=pltpu.CompilerParams(needs_layout_passes=False),
        out_type=jax.ShapeDtypeStruct((2, ACCN, 128), jnp.float32),
        scratch_types=[pltpu.VMEM((NB, BB), jnp.int32),
                       pltpu.VMEM((NB, BB), jnp.int32),
                       pltpu.VMEM((T_EDGE,), jnp.float32),
                       pltpu.VMEM((BB, 128), jnp.float32),
                       pltpu.VMEM((BB, 128), jnp.float32),
                       pltpu.VMEM_SHARED((ACCN, 128), jnp.float32),
                       pltpu.SemaphoreType.DMA,
                       pltpu.SemaphoreType.DMA,
                       pltpu.SemaphoreType.DMA,
                       pltpu.SemaphoreType.DMA],
    )
    def p3(hc, w_all, src3, dst3, zer, out, src2d, dst2d, w_v,
           buf0, buf1, acc, gs0, gs1, ts0, ts1):
        bufs = (buf0, buf1)
        gsems = (gs0, gs1)
        ssems = (ts0, ts1)
        cc = lax.axis_index("c")
        ss = lax.axis_index("s")
        wid = cc * 16 + ss
        pltpu.sync_copy(src3.at[wid], src2d)
        pltpu.sync_copy(dst3.at[wid], dst2d)
        pltpu.sync_copy(w_all.at[pl.ds(wid * T_EDGE, T_EDGE)], w_v)
        pltpu.sync_copy(zer, acc.at[pl.ds(ss * RPTA, RPTA)])
        plsc.subcore_barrier()

        for b in range(NBUF):
            pltpu.async_copy(hc.at[src2d.at[b]], bufs[b], gsems[b])

        def group(g, _):
            for b in range(NBUF):
                j = g * NBUF + b
                pltpu.make_async_copy(hc.at[src2d.at[j]], bufs[b], gsems[b]).wait()

                def scale(r, _2, b=b, j=j):
                    off = j * BB + r
                    bvec = plsc.load_gather(w_v, [jnp.full((16,), off, jnp.int32)])
                    for k in range(8):
                        bufs[b][r, pl.ds(k * 16, 16)] = (
                            bufs[b][r, pl.ds(k * 16, 16)] * bvec)
                    return 0
                lax.fori_loop(0, BB, scale, 0)
                pltpu.async_copy(bufs[b], acc.at[dst2d.at[j]], ssems[b], add=True)
                pltpu.make_async_copy(bufs[b], acc.at[dst2d.at[j]], ssems[b]).wait()

                @pl.when(j + NBUF < NB)
                def _(b=b, j=j):
                    pltpu.async_copy(hc.at[src2d.at[j + NBUF]], bufs[b], gsems[b])
            return 0
        lax.fori_loop(0, NB // NBUF, group, 0)
        plsc.subcore_barrier()
        pltpu.sync_copy(acc.at[pl.ds(ss * RPTA, RPTA)],
                        out.at[cc, pl.ds(ss * RPTA, RPTA)])

    return p3


def _p2b(den_part, H):
    def body(dp_ref, rd_ref):
        rd_ref[...] = (1.0 / (jnp.sum(dp_ref[...], axis=0) + 1e-16))[:, None, :]

    return pl.pallas_call(
        body,
        grid=(1,),
        in_specs=[pl.BlockSpec((NTILES, H, NACC), lambda i: (0, 0, 0))],
        out_specs=pl.BlockSpec((H, 1, NACC), lambda i: (0, 0, 0)),
        out_shape=jax.ShapeDtypeStruct((H, 1, NACC), jnp.float32),
    )(den_part)


def _p4a(op_stack, rden3, b_row, H, O):
    ncol = op_stack.shape[0]
    HO = ncol * 128

    def body(op_ref, rd_ref, b_ref, y_ref, s_ref):
        r = pl.program_id(1)
        agg = op_ref[0, 0] + op_ref[0, 1]
        dsel = rd_ref[0, 0, 0, :]
        y = agg * dsel[:, None] + b_ref[0, :][None, :]
        y_ref[...] = y
        su = jnp.sum(y, axis=0)
        sq = jnp.sum(y * y, axis=0)
        stat = jnp.concatenate(
            [su[None, :], sq[None, :], jnp.zeros((6, 128), jnp.float32)], axis=0)

        @pl.when(r == 0)
        def _():
            s_ref[...] = stat

        @pl.when(r > 0)
        def _():
            s_ref[...] = s_ref[...] + stat

    return pl.pallas_call(
        body,
        grid=(ncol, NRB),
        in_specs=[
            pl.BlockSpec((1, 2, RB, 128), lambda c, r: (c, 0, r, 0)),
            pl.BlockSpec((1, 1, 1, RB), lambda c, r: ((c * 128) // O, r, 0, 0)),
            pl.BlockSpec((1, 128), lambda c, r: (0, c)),
        ],
        out_specs=[
            pl.BlockSpec((RB, 128), lambda c, r: (r, c)),
            pl.BlockSpec((8, 128), lambda c, r: (0, c)),
        ],
        out_shape=[
            jax.ShapeDtypeStruct((N, HO), jnp.float32),
            jax.ShapeDtypeStruct((8, HO), jnp.float32),
        ],
    )(op_stack, rden3, b_row)


def _p4b(y, stats, g_row, be_row):
    HO = y.shape[1]

    def body(y_ref, s_ref, g_ref, be_ref, x_ref):
        mu = s_ref[0, :] / N
        var = s_ref[1, :] / N - mu * mu
        scale = g_ref[0, :] * lax.rsqrt(var + 1e-5)
        xn = (y_ref[...] - mu[None, :]) * scale[None, :] + be_ref[0, :][None, :]
        neg = jnp.exp(jnp.minimum(xn, 0.0)) - 1.0
        x_ref[...] = jnp.where(xn > 0.0, xn, neg)

    return pl.pallas_call(
        body,
        grid=(NRB,),
        in_specs=[
            pl.BlockSpec((RB, HO), lambda r: (r, 0)),
            pl.BlockSpec((8, HO), lambda r: (0, 0)),
            pl.BlockSpec((1, HO), lambda r: (0, 0)),
            pl.BlockSpec((1, HO), lambda r: (0, 0)),
        ],
        out_specs=pl.BlockSpec((RB, HO), lambda r: (r, 0)),
        out_shape=jax.ShapeDtypeStruct((N, HO), jnp.float32),
    )(y, stats, g_row, be_row)


def _k1(h2, Wmu, bmu, Wlv, blv, eps, Wd0, bd0):
    def body(h_ref, wm_ref, bm_ref, wl_ref, bl_ref, e_ref, w0_ref, b0_ref,
             mu_ref, lv_ref, z_ref, a0_ref, s_ref):
        r = pl.program_id(0)
        hb = h_ref[...]
        mu = jnp.dot(hb, wm_ref[...], preferred_element_type=jnp.float32) + bm_ref[0, :][None, :]
        lv = jnp.dot(hb, wl_ref[...], preferred_element_type=jnp.float32) + bl_ref[0, :][None, :]
        z = mu + e_ref[...] * jnp.exp(0.5 * lv)
        a0 = jnp.dot(z, w0_ref[...], preferred_element_type=jnp.float32) + b0_ref[0, :][None, :]
        mu_ref[...] = mu
        lv_ref[...] = lv
        z_ref[...] = z
        a0_ref[...] = a0
        su = jnp.sum(a0, axis=0)
        sq = jnp.sum(a0 * a0, axis=0)
        stat = jnp.concatenate(
            [su[None, :], sq[None, :], jnp.zeros((6, 128), jnp.float32)], axis=0)

        @pl.when(r == 0)
        def _():
            s_ref[...] = stat

        @pl.when(r > 0)
        def _():
            s_ref[...] = s_ref[...] + stat

    return pl.pallas_call(
        body,
        grid=(NRB,),
        in_specs=[
            pl.BlockSpec((RB, 128), lambda r: (r, 0)),
            pl.BlockSpec((128, 32), lambda r: (0, 0)),
            pl.BlockSpec((1, 32), lambda r: (0, 0)),
            pl.BlockSpec((128, 32), lambda r: (0, 0)),
            pl.BlockSpec((1, 32), lambda r: (0, 0)),
            pl.BlockSpec((RB, 32), lambda r: (r, 0)),
            pl.BlockSpec((32, 128), lambda r: (0, 0)),
            pl.BlockSpec((1, 128), lambda r: (0, 0)),
        ],
        out_specs=[
            pl.BlockSpec((RB, 32), lambda r: (r, 0)),
            pl.BlockSpec((RB, 32), lambda r: (r, 0)),
            pl.BlockSpec((RB, 32), lambda r: (r, 0)),
            pl.BlockSpec((RB, 128), lambda r: (r, 0)),
            pl.BlockSpec((8, 128), lambda r: (0, 0)),
        ],
        out_shape=[
            jax.ShapeDtypeStruct((N, 32), jnp.float32),
            jax.ShapeDtypeStruct((N, 32), jnp.float32),
            jax.ShapeDtypeStruct((N, 32), jnp.float32),
            jax.ShapeDtypeStruct((N, 128), jnp.float32),
            jax.ShapeDtypeStruct((8, 128), jnp.float32),
        ],
    )(h2, Wmu, bmu, Wlv, blv, eps, Wd0, bd0)


def _kbn_mm(a, s, g_row, be_row, Wn, bn_, last=False):
    Din = a.shape[1]
    Dout = Wn.shape[1]

    def body(a_ref, s_ref, g_ref, be_ref, w_ref, b_ref, o_ref, so_ref):
        r = pl.program_id(0)
        mu = s_ref[0, :] / N
        var = s_ref[1, :] / N - mu * mu
        scale = g_ref[0, :] * lax.rsqrt(var + 1e-5)
        d = (a_ref[...] - mu[None, :]) * scale[None, :] + be_ref[0, :][None, :]
        d = jnp.maximum(d, 0.0)
        o = jnp.dot(d, w_ref[...], preferred_element_type=jnp.float32) + b_ref[0, :][None, :]
        o_ref[...] = o
        su = jnp.sum(o, axis=0)
        sq = jnp.sum(o * o, axis=0)
        stat = jnp.concatenate(
            [su[None, :], sq[None, :], jnp.zeros((6, Dout), jnp.float32)], axis=0)

        @pl.when(r == 0)
        def _():
            so_ref[...] = stat

        @pl.when(r > 0)
        def _():
            so_ref[...] = so_ref[...] + stat

    return pl.pallas_call(
        body,
        grid=(NRB,),
        in_specs=[
            pl.BlockSpec((RB, Din), lambda r: (r, 0)),
            pl.BlockSpec((8, Din), lambda r: (0, 0)),
            pl.BlockSpec((1, Din), lambda r: (0, 0)),
            pl.BlockSpec((1, Din), lambda r: (0, 0)),
            pl.BlockSpec((Din, Dout), lambda r: (0, 0)),
            pl.BlockSpec((1, Dout), lambda r: (0, 0)),
        ],
        out_specs=[
            pl.BlockSpec((RB, Dout), lambda r: (r, 0)),
            pl.BlockSpec((8, Dout), lambda r: (0, 0)),
        ],
        out_shape=[
            jax.ShapeDtypeStruct((N, Dout), jnp.float32),
            jax.ShapeDtypeStruct((8, Dout), jnp.float32),
        ],
    )(a, s, g_row, be_row, Wn, bn_)


def kernel(x, edge_index, params):
    p = params
    idt = edge_index.dtype
    loop = jnp.arange(N, dtype=idt)
    pad = EPAD - E - N
    src = jnp.concatenate([edge_index[0], loop, jnp.zeros((pad,), idt)]).astype(jnp.int32)
    dst = jnp.concatenate([edge_index[1], loop, jnp.full((pad,), DUMMY, idt)]).astype(jnp.int32)
    src_all = src
    dst_all = dst
    src3 = src.reshape(NTILES, NB, BB)
    dst3 = dst.reshape(NTILES, NB, BB)
    zer = jnp.zeros((RPTA, 128), jnp.float32)
    p2_4 = _make_p2(4)
    p2_1 = _make_p2(1)
    p3 = _make_p3()

    def attn(esed, H):
        es = esed[:, :H]
        ed = esed[:, H:]
        est = jnp.pad(es, ((0, NACC - N), (0, 0))).T.reshape(-1)
        edt = jnp.pad(ed, ((0, NACC - N), (0, 0))).T.reshape(-1)
        p2 = p2_4 if H == 4 else p2_1
        w_e, den_p = p2(est, edt, src_all, dst_all)
        rden3 = _p2b(den_p.reshape(NTILES, H, NACC), H)
        rden4 = rden3.reshape(H, NACC)[:, :N].reshape(H, NRB, 1, RB)
        return w_e.reshape(H, EPAD), rden4

    def gat(xin, W, a_s, a_d, b, H, O):
        h, esed = _p1(xin, W, a_s, a_d, H, O)
        w_resh, rden4 = attn(esed, H)
        chunks = []
        for c in range((H * O) // 128):
            hh = (c * 128) // O
            hc = h[:, c * 128:(c + 1) * 128]
            chunks.append(p3(hc, w_resh[hh], src3, dst3, zer))
        op_stack = jnp.stack(chunks, axis=0)
        return _p4a(op_stack, rden4, b.reshape(1, -1), H, O)

    esed0 = _p1l0(x, p['W0'], p['as0'], p['ad0'], 4, 512)
    w0_resh, rden0 = attn(esed0, 4)
    chunks0 = [p3(x, w0_resh[hh], src3, dst3, zer) for hh in range(4)]
    y0, s0 = _p4l0(jnp.stack(chunks0, axis=0), rden0, p['W0'],
                   p['b0'].reshape(1, -1), 4, 512)
    x1 = _p4b(y0, s0, p['g0'].reshape(1, -1), p['be0'].reshape(1, -1))
    y1, s1 = gat(x1, p['W1'], p['as1'], p['ad1'], p['b1'], 4, 256)
    x2 = _p4b(y1, s1, p['g1'].reshape(1, -1), p['be1'].reshape(1, -1))
    h2, _ = gat(x2, p['W2'], p['as2'], p['ad2'], p['b2'], 1, 128)

    eps = jax.random.normal(jax.random.key(42), (N, 32), dtype=jnp.float32)
    mu, lv, z, a0, st0 = _k1(h2, p['Wmu'], p['bmu'].reshape(1, -1),
                             p['Wlv'], p['blv'].reshape(1, -1), eps,
                             p['Wd0'], p['bd0'].reshape(1, -1))
    a1, st1 = _kbn_mm(a0, st0, p['gd0'].reshape(1, -1), p['bed0'].reshape(1, -1),
                      p['Wd1'], p['bd1'].reshape(1, -1))
    a2, st2 = _kbn_mm(a1, st1, p['gd1'].reshape(1, -1), p['bed1'].reshape(1, -1),
                      p['Wd2'], p['bd2'].reshape(1, -1))
    recon, _ = _kbn_mm(a2, st2, p['gd2'].reshape(1, -1), p['bed2'].reshape(1, -1),
                       p['Wd3'], p['bd3'].reshape(1, -1))
    return (recon, mu, lv, z)


# speedup vs baseline: 1.0341x; 1.0341x over previous
import functools

import jax
import jax.numpy as jnp
from jax import lax
from jax.experimental import pallas as pl
from jax.experimental.pallas import tpu as pltpu
from jax.experimental.pallas import tpu_sc as plsc

N = 10000
G = 128
E = 160000
NACC = 10240
NTILES = 32
BB = 112
NB = 48
NBUF = 2
T_EDGE = NB * BB
EPAD = NTILES * T_EDGE
DUMMY = N
RB = 1000
NRB = N // RB
RPT = NACC // 16
ACCN = 10112
RPTA = ACCN // 16


def _p1(x, W, a_s, a_d, H, O):
    K = x.shape[1]
    HO = H * O

    def body(x_ref, w_ref, as_ref, ad_ref, h_ref, es_ref):
        hb = jnp.dot(x_ref[...], w_ref[...], preferred_element_type=jnp.float32)
        h_ref[...] = hb
        cols = []
        for hh in range(H):
            hs = hb[:, hh * O:(hh + 1) * O]
            cols.append(jnp.sum(hs * as_ref[hh, :][None, :], axis=1))
        for hh in range(H):
            hs = hb[:, hh * O:(hh + 1) * O]
            cols.append(jnp.sum(hs * ad_ref[hh, :][None, :], axis=1))
        es_ref[...] = jnp.stack(cols, axis=1)

    return pl.pallas_call(
        body,
        grid=(NRB,),
        in_specs=[
            pl.BlockSpec((RB, K), lambda r: (r, 0)),
            pl.BlockSpec((K, HO), lambda r: (0, 0)),
            pl.BlockSpec((H, O), lambda r: (0, 0)),
            pl.BlockSpec((H, O), lambda r: (0, 0)),
        ],
        out_specs=[
            pl.BlockSpec((RB, HO), lambda r: (r, 0)),
            pl.BlockSpec((RB, 2 * H), lambda r: (r, 0)),
        ],
        out_shape=[
            jax.ShapeDtypeStruct((N, HO), jnp.float32),
            jax.ShapeDtypeStruct((N, 2 * H), jnp.float32),
        ],
    )(x, W, a_s, a_d)


def _p1l0(x, W, a_s, a_d, H, O):
    def body(x_ref, w_ref, as_ref, ad_ref, es_ref):
        xb = x_ref[...]
        cols = []
        for hh in range(H):
            vs = jnp.dot(w_ref[:, hh * O:(hh + 1) * O], as_ref[hh, :],
                         preferred_element_type=jnp.float32)
            cols.append(jnp.dot(xb, vs, preferred_element_type=jnp.float32))
        for hh in range(H):
            vd = jnp.dot(w_ref[:, hh * O:(hh + 1) * O], ad_ref[hh, :],
                         preferred_element_type=jnp.float32)
            cols.append(jnp.dot(xb, vd, preferred_element_type=jnp.float32))
        es_ref[...] = jnp.stack(cols, axis=1)

    return pl.pallas_call(
        body,
        grid=(NRB,),
        in_specs=[
            pl.BlockSpec((RB, G), lambda r: (r, 0)),
            pl.BlockSpec((G, H * O), lambda r: (0, 0)),
            pl.BlockSpec((H, O), lambda r: (0, 0)),
            pl.BlockSpec((H, O), lambda r: (0, 0)),
        ],
        out_specs=pl.BlockSpec((RB, 2 * H), lambda r: (r, 0)),
        out_shape=jax.ShapeDtypeStruct((N, 2 * H), jnp.float32),
    )(x, W, a_s, a_d)


def _p4l0(op_stack, rden4, W, b_row, H, O):
    def body(op_ref, rd_ref, w_ref, b_ref, y_ref, s_ref):
        r = pl.program_id(1)
        agg = op_ref[0, 0] + op_ref[0, 1]
        z = agg * rd_ref[0, 0, 0, :][:, None]
        y = jnp.dot(z, w_ref[0], preferred_element_type=jnp.float32) + b_ref[0, :][None, :]
        y_ref[...] = y
        su = jnp.sum(y, axis=0)
        sq = jnp.sum(y * y, axis=0)
        stat = jnp.concatenate(
            [su[None, :], sq[None, :], jnp.zeros((6, O), jnp.float32)], axis=0)

        @pl.when(r == 0)
        def _():
            s_ref[...] = stat

        @pl.when(r > 0)
        def _():
            s_ref[...] = s_ref[...] + stat

    return pl.pallas_call(
        body,
        grid=(H, NRB),
        in_specs=[
            pl.BlockSpec((1, 2, RB, G), lambda h, r: (h, 0, r, 0)),
            pl.BlockSpec((1, 1, 1, RB), lambda h, r: (h, r, 0, 0)),
            pl.BlockSpec((1, G, O), lambda h, r: (h, 0, 0)),
            pl.BlockSpec((1, O), lambda h, r: (0, h)),
        ],
        out_specs=[
            pl.BlockSpec((RB, O), lambda h, r: (r, h)),
            pl.BlockSpec((8, O), lambda h, r: (0, h)),
        ],
        out_shape=[
            jax.ShapeDtypeStruct((N, H * O), jnp.float32),
            jax.ShapeDtypeStruct((8, H * O), jnp.float32),
        ],
    )(op_stack, rden4, W.reshape(G, H, O).transpose(1, 0, 2), b_row)


@functools.lru_cache(maxsize=None)
def _make_p2(H):
    mesh = plsc.VectorSubcoreMesh(core_axis_name="c", subcore_axis_name="s")

    @functools.partial(
        pl.kernel, mesh=mesh,
        compiler_params=pltpu.CompilerParams(needs_layout_passes=False),
        out_type=(jax.ShapeDtypeStruct((H * EPAD,), jnp.float32),
                  jax.ShapeDtypeStruct((NTILES * H * NACC,), jnp.float32)),
        scratch_types=[pltpu.VMEM((T_EDGE,), jnp.int32),
                       pltpu.VMEM((T_EDGE,), jnp.int32),
                       pltpu.VMEM((NACC,), jnp.float32),
                       pltpu.VMEM((NACC,), jnp.float32),
                       pltpu.VMEM((T_EDGE,), jnp.float32),
                       pltpu.VMEM((NACC,), jnp.float32)],
    )
    def p2(est, edt, src_all, dst_all, w_out, den_out,
           src_v, dst_v, es_v, ed_v, w_v, den_v):
        wid = lax.axis_index("c") * 16 + lax.axis_index("s")
        pltpu.sync_copy(src_all.at[pl.ds(wid * T_EDGE, T_EDGE)], src_v)
        pltpu.sync_copy(dst_all.at[pl.ds(wid * T_EDGE, T_EDGE)], dst_v)
        for hh in range(H):
            pltpu.sync_copy(est.at[pl.ds(hh * NACC, NACC)], es_v)
            pltpu.sync_copy(edt.at[pl.ds(hh * NACC, NACC)], ed_v)

            def zbody(i, _):
                den_v[pl.ds(i * 16, 16)] = jnp.zeros((16,), jnp.float32)
                return 0
            lax.fori_loop(0, NACC // 16, zbody, 0)

            def ebody(g, _):
                sl = pl.ds(g * 16, 16)
                s16 = src_v[sl]
                d16 = dst_v[sl]
                esg = plsc.load_gather(es_v, [s16])
                edg = plsc.load_gather(ed_v, [d16])
                e = esg + edg
                e = jnp.where(e >= 0.0, e, 0.2 * e)
                wv = jnp.exp(e)
                w_v[sl] = wv
                plsc.addupdate_scatter(den_v, [d16], wv)
                return 0
            lax.fori_loop(0, T_EDGE // 16, ebody, 0)

            pltpu.sync_copy(w_v, w_out.at[pl.ds(hh * EPAD + wid * T_EDGE, T_EDGE)])
            pltpu.sync_copy(den_v, den_out.at[pl.ds((wid * H + hh) * NACC, NACC)])

    return p2


@functools.lru_cache(maxsize=None)
def _make_p3():
    mesh = plsc.VectorSubcoreMesh(core_axis_name="c", subcore_axis_name="s")

    @functools.partial(
        pl.kernel, mesh=mesh,
        compiler_params=pltpu.CompilerParams(needs_layout_passes=False),
        out_type=jax.ShapeDtypeStruct((2, ACCN, 128), jnp.float32),
        scratch_types=[pltpu.VMEM((NB, BB), jnp.int32),
                       pltpu.VMEM((NB, BB), jnp.int32),
                       pltpu.VMEM((T_EDGE,), jnp.float32),
                       pltpu.VMEM((BB, 128), jnp.float32),
                       pltpu.VMEM((BB, 128), jnp.float32),
                       pltpu.VMEM_SHARED((ACCN, 128), jnp.float32),
                       pltpu.SemaphoreType.DMA,
                       pltpu.SemaphoreType.DMA,
                       pltpu.SemaphoreType.DMA,
                       pltpu.SemaphoreType.DMA],
    )
    def p3(hc, w_all, src3, dst3, zer, out, src2d, dst2d, w_v,
           buf0, buf1, acc, gs0, gs1, ts0, ts1):
        bufs = (buf0, buf1)
        gsems = (gs0, gs1)
        ssems = (ts0, ts1)
        cc = lax.axis_index("c")
        ss = lax.axis_index("s")
        wid = cc * 16 + ss
        pltpu.sync_copy(src3.at[wid], src2d)
        pltpu.sync_copy(dst3.at[wid], dst2d)
        pltpu.sync_copy(w_all.at[pl.ds(wid * T_EDGE, T_EDGE)], w_v)
        pltpu.sync_copy(zer, acc.at[pl.ds(ss * RPTA, RPTA)])
        plsc.subcore_barrier()

        for b in range(NBUF):
            pltpu.async_copy(hc.at[src2d.at[b]], bufs[b], gsems[b])

        def group(g, _):
            for b in range(NBUF):
                j = g * NBUF + b
                pltpu.make_async_copy(hc.at[src2d.at[j]], bufs[b], gsems[b]).wait()

                def scale(r, _2, b=b, j=j):
                    off = j * BB + r
                    bvec = plsc.load_gather(w_v, [jnp.full((16,), off, jnp.int32)])
                    for k in range(8):
                        bufs[b][r, pl.ds(k * 16, 16)] = (
                            bufs[b][r, pl.ds(k * 16, 16)] * bvec)
                    return 0
                lax.fori_loop(0, BB, scale, 0)
                pltpu.async_copy(bufs[b], acc.at[dst2d.at[j]], ssems[b], add=True)
                pltpu.make_async_copy(bufs[b], acc.at[dst2d.at[j]], ssems[b]).wait()

                @pl.when(j + NBUF < NB)
                def _(b=b, j=j):
                    pltpu.async_copy(hc.at[src2d.at[j + NBUF]], bufs[b], gsems[b])
            return 0
        lax.fori_loop(0, NB // NBUF, group, 0)
        plsc.subcore_barrier()
        pltpu.sync_copy(acc.at[pl.ds(ss * RPTA, RPTA)],
                        out.at[cc, pl.ds(ss * RPTA, RPTA)])

    return p3


def _p2b(den_part, H):
    def body(dp_ref, rd_ref):
        rd_ref[...] = (1.0 / (jnp.sum(dp_ref[...], axis=0) + 1e-16))[:, None, :]

    return pl.pallas_call(
        body,
        grid=(1,),
        in_specs=[pl.BlockSpec((NTILES, H, NACC), lambda i: (0, 0, 0))],
        out_specs=pl.BlockSpec((H, 1, NACC), lambda i: (0, 0, 0)),
        out_shape=jax.ShapeDtypeStruct((H, 1, NACC), jnp.float32),
    )(den_part)


def _p4a(op_stack, rden3, b_row, H, O):
    ncol = op_stack.shape[0]
    HO = ncol * 128

    def body(op_ref, rd_ref, b_ref, y_ref, s_ref):
        r = pl.program_id(1)
        agg = op_ref[0, 0] + op_ref[0, 1]
        dsel = rd_ref[0, 0, 0, :]
        y = agg * dsel[:, None] + b_ref[0, :][None, :]
        y_ref[...] = y
        su = jnp.sum(y, axis=0)
        sq = jnp.sum(y * y, axis=0)
        stat = jnp.concatenate(
            [su[None, :], sq[None, :], jnp.zeros((6, 128), jnp.float32)], axis=0)

        @pl.when(r == 0)
        def _():
            s_ref[...] = stat

        @pl.when(r > 0)
        def _():
            s_ref[...] = s_ref[...] + stat

    return pl.pallas_call(
        body,
        grid=(ncol, NRB),
        in_specs=[
            pl.BlockSpec((1, 2, RB, 128), lambda c, r: (c, 0, r, 0)),
            pl.BlockSpec((1, 1, 1, RB), lambda c, r: ((c * 128) // O, r, 0, 0)),
            pl.BlockSpec((1, 128), lambda c, r: (0, c)),
        ],
        out_specs=[
            pl.BlockSpec((RB, 128), lambda c, r: (r, c)),
            pl.BlockSpec((8, 128), lambda c, r: (0, c)),
        ],
        out_shape=[
            jax.ShapeDtypeStruct((N, HO), jnp.float32),
            jax.ShapeDtypeStruct((8, HO), jnp.float32),
        ],
    )(op_stack, rden3, b_row)


def _p4b(y, stats, g_row, be_row):
    HO = y.shape[1]

    def body(y_ref, s_ref, g_ref, be_ref, x_ref):
        mu = s_ref[0, :] / N
        var = s_ref[1, :] / N - mu * mu
        scale = g_ref[0, :] * lax.rsqrt(var + 1e-5)
        xn = (y_ref[...] - mu[None, :]) * scale[None, :] + be_ref[0, :][None, :]
        neg = jnp.exp(jnp.minimum(xn, 0.0)) - 1.0
        x_ref[...] = jnp.where(xn > 0.0, xn, neg)

    return pl.pallas_call(
        body,
        grid=(NRB,),
        in_specs=[
            pl.BlockSpec((RB, HO), lambda r: (r, 0)),
            pl.BlockSpec((8, HO), lambda r: (0, 0)),
            pl.BlockSpec((1, HO), lambda r: (0, 0)),
            pl.BlockSpec((1, HO), lambda r: (0, 0)),
        ],
        out_specs=pl.BlockSpec((RB, HO), lambda r: (r, 0)),
        out_shape=jax.ShapeDtypeStruct((N, HO), jnp.float32),
    )(y, stats, g_row, be_row)


def _k1(h2, Wmu, bmu, Wlv, blv, eps, Wd0, bd0):
    def body(h_ref, wm_ref, bm_ref, wl_ref, bl_ref, e_ref, w0_ref, b0_ref,
             mu_ref, lv_ref, z_ref, a0_ref, s_ref):
        r = pl.program_id(0)
        hb = h_ref[...]
        mu = jnp.dot(hb, wm_ref[...], preferred_element_type=jnp.float32) + bm_ref[0, :][None, :]
        lv = jnp.dot(hb, wl_ref[...], preferred_element_type=jnp.float32) + bl_ref[0, :][None, :]
        z = mu + e_ref[...] * jnp.exp(0.5 * lv)
        a0 = jnp.dot(z, w0_ref[...], preferred_element_type=jnp.float32) + b0_ref[0, :][None, :]
        mu_ref[...] = mu
        lv_ref[...] = lv
        z_ref[...] = z
        a0_ref[...] = a0
        su = jnp.sum(a0, axis=0)
        sq = jnp.sum(a0 * a0, axis=0)
        stat = jnp.concatenate(
            [su[None, :], sq[None, :], jnp.zeros((6, 128), jnp.float32)], axis=0)

        @pl.when(r == 0)
        def _():
            s_ref[...] = stat

        @pl.when(r > 0)
        def _():
            s_ref[...] = s_ref[...] + stat

    return pl.pallas_call(
        body,
        grid=(NRB,),
        in_specs=[
            pl.BlockSpec((RB, 128), lambda r: (r, 0)),
            pl.BlockSpec((128, 32), lambda r: (0, 0)),
            pl.BlockSpec((1, 32), lambda r: (0, 0)),
            pl.BlockSpec((128, 32), lambda r: (0, 0)),
            pl.BlockSpec((1, 32), lambda r: (0, 0)),
            pl.BlockSpec((RB, 32), lambda r: (r, 0)),
            pl.BlockSpec((32, 128), lambda r: (0, 0)),
            pl.BlockSpec((1, 128), lambda r: (0, 0)),
        ],
        out_specs=[
            pl.BlockSpec((RB, 32), lambda r: (r, 0)),
            pl.BlockSpec((RB, 32), lambda r: (r, 0)),
            pl.BlockSpec((RB, 32), lambda r: (r, 0)),
            pl.BlockSpec((RB, 128), lambda r: (r, 0)),
            pl.BlockSpec((8, 128), lambda r: (0, 0)),
        ],
        out_shape=[
            jax.ShapeDtypeStruct((N, 32), jnp.float32),
            jax.ShapeDtypeStruct((N, 32), jnp.float32),
            jax.ShapeDtypeStruct((N, 32), jnp.float32),
            jax.ShapeDtypeStruct((N, 128), jnp.float32),
            jax.ShapeDtypeStruct((8, 128), jnp.float32),
        ],
    )(h2, Wmu, bmu, Wlv, blv, eps, Wd0, bd0)


def _kbn_mm(a, s, g_row, be_row, Wn, bn_, last=False):
    Din = a.shape[1]
    Dout = Wn.shape[1]

    def body(a_ref, s_ref, g_ref, be_ref, w_ref, b_ref, o_ref, so_ref):
        r = pl.program_id(0)
        mu = s_ref[0, :] / N
        var = s_ref[1, :] / N - mu * mu
        scale = g_ref[0, :] * lax.rsqrt(var + 1e-5)
        d = (a_ref[...] - mu[None, :]) * scale[None, :] + be_ref[0, :][None, :]
        d = jnp.maximum(d, 0.0)
        o = jnp.dot(d, w_ref[...], preferred_element_type=jnp.float32) + b_ref[0, :][None, :]
        o_ref[...] = o
        su = jnp.sum(o, axis=0)
        sq = jnp.sum(o * o, axis=0)
        stat = jnp.concatenate(
            [su[None, :], sq[None, :], jnp.zeros((6, Dout), jnp.float32)], axis=0)

        @pl.when(r == 0)
        def _():
            so_ref[...] = stat

        @pl.when(r > 0)
        def _():
            so_ref[...] = so_ref[...] + stat

    return pl.pallas_call(
        body,
        grid=(NRB,),
        in_specs=[
            pl.BlockSpec((RB, Din), lambda r: (r, 0)),
            pl.BlockSpec((8, Din), lambda r: (0, 0)),
            pl.BlockSpec((1, Din), lambda r: (0, 0)),
            pl.BlockSpec((1, Din), lambda r: (0, 0)),
            pl.BlockSpec((Din, Dout), lambda r: (0, 0)),
            pl.BlockSpec((1, Dout), lambda r: (0, 0)),
        ],
        out_specs=[
            pl.BlockSpec((RB, Dout), lambda r: (r, 0)),
            pl.BlockSpec((8, Dout), lambda r: (0, 0)),
        ],
        out_shape=[
            jax.ShapeDtypeStruct((N, Dout), jnp.float32),
            jax.ShapeDtypeStruct((8, Dout), jnp.float32),
        ],
    )(a, s, g_row, be_row, Wn, bn_)


def kernel(x, edge_index, params):
    p = params
    idt = edge_index.dtype
    loop = jnp.arange(N, dtype=idt)
    pad = EPAD - E - N
    src = jnp.concatenate([edge_index[0], loop, jnp.zeros((pad,), idt)]).astype(jnp.int32)
    dst = jnp.concatenate([edge_index[1], loop, jnp.full((pad,), DUMMY, idt)]).astype(jnp.int32)
    src_all = src
    dst_all = dst
    src3 = src.reshape(NTILES, NB, BB)
    dst3 = dst.reshape(NTILES, NB, BB)
    zer = jnp.zeros((RPTA, 128), jnp.float32)
    p2_4 = _make_p2(4)
    p2_1 = _make_p2(1)
    p3 = _make_p3()

    def attn(esed, H):
        es = esed[:, :H]
        ed = esed[:, H:]
        est = jnp.pad(es, ((0, NACC - N), (0, 0))).T.reshape(-1)
        edt = jnp.pad(ed, ((0, NACC - N), (0, 0))).T.reshape(-1)
        p2 = p2_4 if H == 4 else p2_1
        w_e, den_p = p2(est, edt, src_all, dst_all)
        rden3 = _p2b(den_p.reshape(NTILES, H, NACC), H)
        rden4 = rden3.reshape(H, NACC)[:, :N].reshape(H, NRB, 1, RB)
        return w_e.reshape(H, EPAD), rden4

    def gat(xin, W, a_s, a_d, b, H, O):
        h, esed = _p1(xin, W, a_s, a_d, H, O)
        w_resh, rden4 = attn(esed, H)
        chunks = []
        for c in range((H * O) // 128):
            hh = (c * 128) // O
            hc = h[:, c * 128:(c + 1) * 128]
            chunks.append(p3(hc, w_resh[hh], src3, dst3, zer))
        op_stack = jnp.stack(chunks, axis=0)
        return _p4a(op_stack, rden4, b.reshape(1, -1), H, O)

    esed0 = _p1l0(x, p['W0'], p['as0'], p['ad0'], 4, 512)
    w0_resh, rden0 = attn(esed0, 4)
    chunks0 = [p3(x, w0_resh[hh], src3, dst3, zer) for hh in range(4)]
    y0, s0 = _p4l0(jnp.stack(chunks0, axis=0), rden0, p['W0'],
                   p['b0'].reshape(1, -1), 4, 512)
    x1 = _p4b(y0, s0, p['g0'].reshape(1, -1), p['be0'].reshape(1, -1))
    y1, s1 = gat(x1, p['W1'], p['as1'], p['ad1'], p['b1'], 4, 256)
    x2 = _p4b(y1, s1, p['g1'].reshape(1, -1), p['be1'].reshape(1, -1))
    h2, _ = gat(x2, p['W2'], p['as2'], p['ad2'], p['b2'], 1, 128)

    eps = jax.random.normal(jax.random.key(42), (N, 32), dtype=jnp.float32)
    mu, lv, z, a0, st0 = _k1(h2, p['Wmu'], p['bmu'].reshape(1, -1),
                             p['Wlv'], p['blv'].reshape(1, -1), eps,
                             p['Wd0'], p['bd0'].reshape(1, -1))
    a1, st1 = _kbn_mm(a0, st0, p['gd0'].reshape(1, -1), p['bed0'].reshape(1, -1),
                      p['Wd1'], p['bd1'].reshape(1, -1))
    a2, st2 = _kbn_mm(a1, st1, p['gd1'].reshape(1, -1), p['bed1'].reshape(1, -1),
                      p['Wd2'], p['bd2'].reshape(1, -1))
    recon, _ = _kbn_mm(a2, st2, p['gd2'].reshape(1, -1), p['bed2'].reshape(1, -1),
                       p['Wd3'], p['bd3'].reshape(1, -1))
    return (recon, mu, lv, z)
